# Initial kernel scaffold; baseline (speedup 1.0000x reference)
import jax
import jax.numpy as jnp
from jax.experimental import pallas as pl


def kernel(x, edge_index, W1, b1, W2, b2, Wl, bl):
    raise NotImplementedError("write your pallas kernel here")


# speedup vs baseline: 1.0000x; 1.0000x over previous
import jax
import jax.numpy as jnp
from jax.experimental import pallas as pl


def _gcn_conv(x, src, dst, W, b):
    n = x.shape[0]
    loop = jnp.arange(n, dtype=src.dtype)
    s = jnp.concatenate([src, loop])
    d = jnp.concatenate([dst, loop])
    deg = jnp.zeros((n,), dtype=x.dtype).at[d].add(1.0)
    dinv = jnp.where(deg > 0, jax.lax.rsqrt(deg), 0.0)
    norm = dinv[s] * dinv[d]
    h = x @ W
    msg = h[s] * norm[:, None]
    out = jnp.zeros((n, h.shape[1]), dtype=x.dtype).at[d].add(msg)
    return out + b


def kernel(x, edge_index, W1, b1, W2, b2, Wl, bl):
    src = edge_index[0]
    dst = edge_index[1]
    h = jax.nn.relu(_gcn_conv(x, src, dst, W1, b1))
    h = jax.nn.relu(_gcn_conv(h, src, dst, W2, b2))
    return h @ Wl + bl


# speedup vs baseline: 40.2256x; 40.2239x over previous
import functools

import jax
import jax.numpy as jnp
from jax import lax
from jax.experimental import pallas as pl
from jax.experimental.pallas import tpu as pltpu
from jax.experimental.pallas import tpu_sc as plsc

N = 10000
NPAD = 10240
E = 320000
NC = 2
NS = 16
NW = NC * NS
EW = 10240
EPAD = NW * EW
C = 128
NJ = EW // C
RPT = NPAD // NS
D = 16

_MESH = plsc.VectorSubcoreMesh(core_axis_name="c", subcore_axis_name="s")


@functools.partial(
    pl.kernel,
    out_type=jax.ShapeDtypeStruct((NC, NPAD), jnp.float32),
    mesh=_MESH,
    scratch_types=[
        pltpu.VMEM((NJ, C), jnp.int32),
        pltpu.VMEM((C,), jnp.float32),
        pltpu.MemorySpace.VMEM_SHARED((NPAD,), jnp.float32),
    ],
)
def _deg_kernel(dst_hbm, zeros_hbm, deg_out, didx, ones_v, acc):
    cid = lax.axis_index("c")
    sid = lax.axis_index("s")
    wid = sid * NC + cid
    pltpu.sync_copy(dst_hbm.at[wid], didx)
    for i in range(C // 16):
        ones_v[pl.ds(i * 16, 16)] = jnp.ones((16,), jnp.float32)
    pltpu.sync_copy(zeros_hbm.at[pl.ds(sid * RPT, RPT)],
                    acc.at[pl.ds(sid * RPT, RPT)])
    plsc.subcore_barrier()

    def body(j, carry):
        pltpu.sync_copy(ones_v, acc.at[didx.at[j]], add=True)
        return carry

    lax.fori_loop(0, NJ, body, 0)
    plsc.subcore_barrier()
    pltpu.sync_copy(acc.at[pl.ds(sid * RPT, RPT)],
                    deg_out.at[cid, pl.ds(sid * RPT, RPT)])


@functools.partial(
    pl.kernel,
    out_type=jax.ShapeDtypeStruct((NC, NPAD, D), jnp.float32),
    mesh=_MESH,
    scratch_types=[
        pltpu.VMEM((NJ, C), jnp.int32),
        pltpu.VMEM((NJ, C), jnp.int32),
        pltpu.VMEM((2, C, D), jnp.float32),
        pltpu.SemaphoreType.DMA((2,)),
        pltpu.MemorySpace.VMEM_SHARED((NPAD, D), jnp.float32),
    ],
    compiler_params=pltpu.CompilerParams(use_tc_tiling_on_sc=False),
)
def _agg_kernel(g_hbm, src_hbm, dst_hbm, zeros_hbm, part_out,
                sidx, didx, rows, sems, acc):
    cid = lax.axis_index("c")
    sid = lax.axis_index("s")
    wid = sid * NC + cid
    pltpu.sync_copy(src_hbm.at[wid], sidx)
    pltpu.sync_copy(dst_hbm.at[wid], didx)
    pltpu.sync_copy(zeros_hbm.at[pl.ds(sid * RPT, RPT)],
                    acc.at[pl.ds(sid * RPT, RPT)])
    plsc.subcore_barrier()

    for b in range(2):
        pltpu.async_copy(g_hbm.at[sidx.at[b]], rows.at[b], sems.at[b])

    def body(i, carry):
        for b in range(2):
            j = i * 2 + b
            pltpu.make_async_copy(g_hbm.at[sidx.at[0]], rows.at[b],
                                  sems.at[b]).wait()
            pltpu.sync_copy(rows.at[b], acc.at[didx.at[j]], add=True)

            @pl.when(j + 2 < NJ)
            def _():
                pltpu.async_copy(g_hbm.at[sidx.at[j + 2]], rows.at[b],
                                 sems.at[b])
        return carry

    lax.fori_loop(0, NJ // 2, body, 0)
    plsc.subcore_barrier()
    pltpu.sync_copy(acc.at[pl.ds(sid * RPT, RPT)],
                    part_out.at[cid, pl.ds(sid * RPT, RPT)])


def _tc1_body(x_ref, w1_ref, deg_ref, g1_ref, dinv_ref):
    deg = jnp.sum(deg_ref[...], axis=0) + 1.0
    dinv = lax.rsqrt(deg)[:, None]
    h1 = jnp.dot(x_ref[...], w1_ref[...], preferred_element_type=jnp.float32)
    g1_ref[...] = h1 * dinv
    dinv_ref[...] = dinv


def _tc2_body(s1p_ref, g1_ref, dinv_ref, b1_ref, w2_ref, g2_ref):
    dinv = dinv_ref[...]
    s1 = s1p_ref[0] + s1p_ref[1] + g1_ref[...]
    a1 = jnp.maximum(s1 * dinv + b1_ref[...], 0.0)
    h2 = jnp.dot(a1, w2_ref[...], preferred_element_type=jnp.float32)
    g2_ref[...] = h2 * dinv


def _tc3_body(s2p_ref, g2_ref, dinv_ref, b2_ref, wl_ref, bl_ref, out_ref):
    dinv = dinv_ref[...]
    s2 = s2p_ref[0] + s2p_ref[1] + g2_ref[...]
    a2 = jnp.maximum(s2 * dinv + b2_ref[...], 0.0)
    out_ref[...] = (
        jnp.dot(a2, wl_ref[...], preferred_element_type=jnp.float32)
        + bl_ref[...]
    )


def _tc_call(body, out_shapes, *args):
    return pl.pallas_call(
        body,
        out_shape=[jax.ShapeDtypeStruct(s, jnp.float32) for s in out_shapes],
    )(*args)


def kernel(x, edge_index, W1, b1, W2, b2, Wl, bl):
    src = edge_index[0].astype(jnp.int32)
    dst = edge_index[1].astype(jnp.int32)
    npad_e = EPAD - E
    src_p = jnp.concatenate([src, jnp.full((npad_e,), N, jnp.int32)])
    dst_p = jnp.concatenate([dst, jnp.full((npad_e,), NPAD - 1, jnp.int32)])
    src3 = src_p.reshape(NW, NJ, C)
    dst3 = dst_p.reshape(NW, NJ, C)

    x_pad = jnp.pad(x, ((0, NPAD - N), (0, 0)))
    zrow = jnp.zeros((NPAD,), jnp.float32)
    zero16 = jnp.zeros((NPAD, D), jnp.float32)
    w2p = jnp.pad(W2, ((0, 0), (0, D - W2.shape[1])))
    b2p = jnp.pad(b2, (0, D - b2.shape[0]))[None, :]
    wlp = jnp.pad(Wl, ((0, D - Wl.shape[0]), (0, 0)))

    deg2 = _deg_kernel(dst3, zrow)
    g1, dinv = _tc_call(_tc1_body, [(NPAD, D), (NPAD, 1)],
                        x_pad, W1, deg2)
    s1p = _agg_kernel(g1, src3, dst3, zero16)
    (g2,) = _tc_call(_tc2_body, [(NPAD, D)],
                     s1p, g1, dinv, b1[None, :], w2p)
    s2p = _agg_kernel(g2, src3, dst3, zero16)
    (out,) = _tc_call(_tc3_body, [(NPAD, 1)],
                      s2p, g2, dinv, b2p, wlp, bl[None, :])
    return out[:N]


# speedup vs baseline: 62.6866x; 1.5584x over previous
import functools

import jax
import jax.numpy as jnp
from jax import lax
from jax.experimental import pallas as pl
from jax.experimental.pallas import tpu as pltpu
from jax.experimental.pallas import tpu_sc as plsc

N = 10000
NPAD = 10240
E = 320000
NC = 2
NS = 16
NW = NC * NS
EW = E // NW
C = 80
NJ = EW // C
NBUF = 5
RPT = NPAD // NS
D = 16

_MESH = plsc.VectorSubcoreMesh(core_axis_name="c", subcore_axis_name="s")
_SC_PARAMS = pltpu.CompilerParams(use_tc_tiling_on_sc=False)


@functools.partial(
    pl.kernel,
    out_type=jax.ShapeDtypeStruct((NC, NPAD), jnp.float32),
    mesh=_MESH,
    scratch_types=[
        pltpu.VMEM((NJ, C), jnp.int32),
        pltpu.VMEM((C,), jnp.float32),
        pltpu.SemaphoreType.DMA,
        pltpu.MemorySpace.VMEM_SHARED((NPAD,), jnp.float32),
    ],
    compiler_params=_SC_PARAMS,
)
def _deg_kernel(edges_hbm, zeros_hbm, deg_out, didx, ones_v, sem, acc):
    cid = lax.axis_index("c")
    sid = lax.axis_index("s")
    wid = sid * NC + cid
    pltpu.sync_copy(edges_hbm.at[1, wid], didx)
    for i in range(C // 16):
        ones_v[pl.ds(i * 16, 16)] = jnp.ones((16,), jnp.float32)
    pltpu.sync_copy(zeros_hbm.at[pl.ds(sid * RPT, RPT)],
                    acc.at[pl.ds(sid * RPT, RPT)])
    plsc.subcore_barrier()

    def body(j, carry):
        pltpu.async_copy(ones_v, acc.at[didx.at[j]], sem, add=True)

        @pl.when(j >= NBUF)
        def _():
            pltpu.make_async_copy(ones_v, acc.at[didx.at[0]], sem).wait()
        return carry

    lax.fori_loop(0, NJ, body, 0)
    for _ in range(NBUF):
        pltpu.make_async_copy(ones_v, acc.at[didx.at[0]], sem).wait()
    plsc.subcore_barrier()
    pltpu.sync_copy(acc.at[pl.ds(sid * RPT, RPT)],
                    deg_out.at[cid, pl.ds(sid * RPT, RPT)])


@functools.partial(
    pl.kernel,
    out_type=jax.ShapeDtypeStruct((NC, NPAD, D), jnp.float32),
    mesh=_MESH,
    scratch_types=[
        pltpu.VMEM((NJ, C), jnp.int32),
        pltpu.VMEM((NJ, C), jnp.int32),
        pltpu.VMEM((NBUF, C, D), jnp.float32),
        pltpu.SemaphoreType.DMA((NBUF,)),
        pltpu.SemaphoreType.DMA((NBUF,)),
        pltpu.MemorySpace.VMEM_SHARED((NPAD, D), jnp.float32),
    ],
    compiler_params=_SC_PARAMS,
)
def _agg_kernel(g_hbm, edges_hbm, zeros_hbm, part_out,
                sidx, didx, rows, gsem, ssem, acc):
    cid = lax.axis_index("c")
    sid = lax.axis_index("s")
    wid = sid * NC + cid
    pltpu.sync_copy(edges_hbm.at[0, wid], sidx)
    pltpu.sync_copy(edges_hbm.at[1, wid], didx)
    pltpu.sync_copy(zeros_hbm.at[pl.ds(sid * RPT, RPT)],
                    acc.at[pl.ds(sid * RPT, RPT)])
    plsc.subcore_barrier()

    for b in range(NBUF):
        pltpu.async_copy(g_hbm.at[sidx.at[b]], rows.at[b], gsem.at[b])

    def body(i, carry):
        for b in range(NBUF):
            j = i * NBUF + b
            pltpu.make_async_copy(g_hbm.at[sidx.at[0]], rows.at[b],
                                  gsem.at[b]).wait()
            pltpu.async_copy(rows.at[b], acc.at[didx.at[j]], ssem.at[b],
                             add=True)
        for b in range(NBUF):
            j = i * NBUF + b
            pltpu.make_async_copy(rows.at[b], acc.at[didx.at[0]],
                                  ssem.at[b]).wait()

            @pl.when(j + NBUF < NJ)
            def _():
                pltpu.async_copy(g_hbm.at[sidx.at[j + NBUF]], rows.at[b],
                                 gsem.at[b])
        return carry

    lax.fori_loop(0, NJ // NBUF, body, 0)
    plsc.subcore_barrier()
    pltpu.sync_copy(acc.at[pl.ds(sid * RPT, RPT)],
                    part_out.at[cid, pl.ds(sid * RPT, RPT)])


def _tc1_body(x_ref, w1_ref, deg_ref, g1_ref, dinv_ref):
    deg = jnp.sum(deg_ref[...], axis=0) + 1.0
    dinv = lax.rsqrt(deg)[:, None]
    h1 = jnp.dot(x_ref[...], w1_ref[...], preferred_element_type=jnp.float32)
    g1_ref[...] = h1 * dinv
    dinv_ref[...] = dinv


def _tc2_body(s1p_ref, g1_ref, dinv_ref, b1_ref, w2_ref, g2_ref):
    dinv = dinv_ref[...]
    s1 = s1p_ref[0] + s1p_ref[1] + g1_ref[...]
    a1 = jnp.maximum(s1 * dinv + b1_ref[...], 0.0)
    h2 = jnp.dot(a1, w2_ref[...], preferred_element_type=jnp.float32)
    g2_ref[...] = h2 * dinv


def _tc3_body(s2p_ref, g2_ref, dinv_ref, b2_ref, wl_ref, bl_ref, out_ref):
    dinv = dinv_ref[...]
    s2 = s2p_ref[0] + s2p_ref[1] + g2_ref[...]
    a2 = jnp.maximum(s2 * dinv + b2_ref[...], 0.0)
    out_ref[...] = (
        jnp.dot(a2, wl_ref[...], preferred_element_type=jnp.float32)
        + bl_ref[...]
    )


def _tc_call(body, out_shapes, *args):
    return pl.pallas_call(
        body,
        out_shape=[jax.ShapeDtypeStruct(s, jnp.float32) for s in out_shapes],
    )(*args)


def kernel(x, edge_index, W1, b1, W2, b2, Wl, bl):
    edges = edge_index.astype(jnp.int32).reshape(2, NW, NJ, C)
    x_pad = jnp.pad(x, ((0, NPAD - N), (0, 0)))
    zrow = jnp.zeros((NPAD,), jnp.float32)
    zero16 = jnp.zeros((NPAD, D), jnp.float32)
    w2p = jnp.pad(W2, ((0, 0), (0, D - W2.shape[1])))
    b2p = jnp.pad(b2, (0, D - b2.shape[0]))[None, :]
    wlp = jnp.pad(Wl, ((0, D - Wl.shape[0]), (0, 0)))

    deg2 = _deg_kernel(edges, zrow)
    g1, dinv = _tc_call(_tc1_body, [(NPAD, D), (NPAD, 1)],
                        x_pad, W1, deg2)
    s1p = _agg_kernel(g1, edges, zero16)
    (g2,) = _tc_call(_tc2_body, [(NPAD, D)],
                     s1p, g1, dinv, b1[None, :], w2p)
    s2p = _agg_kernel(g2, edges, zero16)
    (out,) = _tc_call(_tc3_body, [(NPAD, 1)],
                      s2p, g2, dinv, b2p, wlp, bl[None, :])
    return out[:N]


# speedup vs baseline: 75.7884x; 1.2090x over previous
import functools

import jax
import jax.numpy as jnp
from jax import lax
from jax.experimental import pallas as pl
from jax.experimental.pallas import tpu as pltpu
from jax.experimental.pallas import tpu_sc as plsc

N = 10000
NPAD = 10240
E = 320000
NC = 2
NS = 16
NW = NC * NS
EW = E // NW
C = 80
NJ = EW // C
NBUF = 5
RPT = NPAD // NS
RPW = NPAD // NW
D = 16

_MESH = plsc.VectorSubcoreMesh(core_axis_name="c", subcore_axis_name="s")
_SC_PARAMS = pltpu.CompilerParams(use_tc_tiling_on_sc=False,
                                  needs_layout_passes=False)


@functools.partial(
    pl.kernel,
    out_type=jax.ShapeDtypeStruct((NC, NPAD), jnp.float32),
    mesh=_MESH,
    scratch_types=[
        pltpu.VMEM((NJ, C), jnp.int32),
        pltpu.VMEM((C,), jnp.float32),
        pltpu.SemaphoreType.DMA,
        pltpu.MemorySpace.VMEM_SHARED((NPAD,), jnp.float32),
    ],
    compiler_params=_SC_PARAMS,
)
def _deg_kernel(edges_hbm, zeros_hbm, deg_out, didx, ones_v, sem, acc):
    cid = lax.axis_index("c")
    sid = lax.axis_index("s")
    wid = sid * NC + cid
    pltpu.sync_copy(edges_hbm.at[1, wid], didx)
    for i in range(C // 16):
        ones_v[pl.ds(i * 16, 16)] = jnp.ones((16,), jnp.float32)
    pltpu.sync_copy(zeros_hbm.at[pl.ds(sid * RPT, RPT)],
                    acc.at[pl.ds(sid * RPT, RPT)])
    plsc.subcore_barrier()

    def body(j, carry):
        pltpu.async_copy(ones_v, acc.at[didx.at[j]], sem, add=True)

        @pl.when(j >= NBUF)
        def _():
            pltpu.make_async_copy(ones_v, acc.at[didx.at[0]], sem).wait()
        return carry

    lax.fori_loop(0, NJ, body, 0)
    for _ in range(NBUF):
        pltpu.make_async_copy(ones_v, acc.at[didx.at[0]], sem).wait()
    plsc.subcore_barrier()
    pltpu.sync_copy(acc.at[pl.ds(sid * RPT, RPT)],
                    deg_out.at[cid, pl.ds(sid * RPT, RPT)])


def _edge_ring(g_src, edges_hbm, sidx, didx, rows, gsem, ssem, acc, wid):
    pltpu.sync_copy(edges_hbm.at[0, wid], sidx)
    pltpu.sync_copy(edges_hbm.at[1, wid], didx)

    for b in range(NBUF):
        pltpu.async_copy(g_src.at[sidx.at[b]], rows.at[b], gsem.at[b])

    def body(i, carry):
        for b in range(NBUF):
            j = i * NBUF + b
            pltpu.make_async_copy(g_src.at[sidx.at[0]], rows.at[b],
                                  gsem.at[b]).wait()
            pltpu.async_copy(rows.at[b], acc.at[didx.at[j]], ssem.at[b],
                             add=True)
        for b in range(NBUF):
            j = i * NBUF + b
            pltpu.make_async_copy(rows.at[b], acc.at[didx.at[0]],
                                  ssem.at[b]).wait()

            @pl.when(j + NBUF < NJ)
            def _():
                pltpu.async_copy(g_src.at[sidx.at[j + NBUF]], rows.at[b],
                                 gsem.at[b])
        return carry

    lax.fori_loop(0, NJ // NBUF, body, 0)


@functools.partial(
    pl.kernel,
    out_type=jax.ShapeDtypeStruct((NC, NPAD, D), jnp.float32),
    mesh=_MESH,
    scratch_types=[
        pltpu.VMEM((NJ, C), jnp.int32),
        pltpu.VMEM((NJ, C), jnp.int32),
        pltpu.VMEM((NBUF, C, D), jnp.float32),
        pltpu.SemaphoreType.DMA((NBUF,)),
        pltpu.SemaphoreType.DMA((NBUF,)),
        pltpu.MemorySpace.VMEM_SHARED((NPAD, D), jnp.float32),
    ],
    compiler_params=_SC_PARAMS,
)
def _agg1_kernel(g_hbm, edges_hbm, zeros_hbm, part_out,
                 sidx, didx, rows, gsem, ssem, acc):
    cid = lax.axis_index("c")
    sid = lax.axis_index("s")
    wid = sid * NC + cid
    pltpu.sync_copy(zeros_hbm.at[pl.ds(sid * RPT, RPT)],
                    acc.at[pl.ds(sid * RPT, RPT)])
    plsc.subcore_barrier()
    _edge_ring(g_hbm, edges_hbm, sidx, didx, rows, gsem, ssem, acc, wid)
    plsc.subcore_barrier()
    pltpu.sync_copy(acc.at[pl.ds(sid * RPT, RPT)],
                    part_out.at[cid, pl.ds(sid * RPT, RPT)])


@functools.partial(
    pl.kernel,
    out_type=[
        jax.ShapeDtypeStruct((NC, NPAD, D), jnp.float32),
        jax.ShapeDtypeStruct((NPAD, D), jnp.float32),
    ],
    mesh=_MESH,
    scratch_types=[
        pltpu.VMEM((NJ, C), jnp.int32),
        pltpu.VMEM((NJ, C), jnp.int32),
        pltpu.VMEM((NBUF, C, D), jnp.float32),
        pltpu.VMEM((RPT, D), jnp.float32),
        pltpu.VMEM((RPT, D), jnp.float32),
        pltpu.VMEM((RPT, D), jnp.float32),
        pltpu.VMEM((RPT,), jnp.float32),
        pltpu.VMEM((16,), jnp.float32),
        pltpu.SemaphoreType.DMA((NBUF,)),
        pltpu.SemaphoreType.DMA((NBUF,)),
        pltpu.MemorySpace.VMEM_SHARED((NPAD, D), jnp.float32),
        pltpu.MemorySpace.VMEM_SHARED((NPAD, D), jnp.float32),
    ],
    compiler_params=_SC_PARAMS,
)
def _agg2_kernel(s1p_hbm, g1_hbm, dinv_hbm, b1_hbm, edges_hbm, zeros_hbm,
                 part_out, u_out,
                 sidx, didx, rows, pa_v, pb_v, u_v, dinv_v, b1_v,
                 gsem, ssem, utab, acc):
    cid = lax.axis_index("c")
    sid = lax.axis_index("s")
    wid = sid * NC + cid
    r0 = sid * RPT
    pltpu.sync_copy(zeros_hbm.at[pl.ds(r0, RPT)], acc.at[pl.ds(r0, RPT)])
    pltpu.sync_copy(s1p_hbm.at[0, pl.ds(r0, RPT)], pa_v)
    pltpu.sync_copy(s1p_hbm.at[1, pl.ds(r0, RPT)], pb_v)
    pltpu.sync_copy(g1_hbm.at[pl.ds(r0, RPT)], u_v)
    pltpu.sync_copy(dinv_hbm.at[pl.ds(r0, RPT)], dinv_v)
    pltpu.sync_copy(b1_hbm, b1_v)

    def ugrp(i, carry):
        dv = dinv_v[pl.ds(i * 16, 16)]
        for k in range(16):
            r = i * 16 + k
            d = lax.broadcast_in_dim(dv[k], (16,), ())
            s1 = pa_v[r] + pb_v[r] + u_v[r]
            u_v[r] = d * jnp.maximum(d * s1 + b1_v[...], 0.0)
        return carry

    lax.fori_loop(0, RPT // 16, ugrp, 0)
    pltpu.sync_copy(u_v, utab.at[pl.ds(r0, RPT)])

    @pl.when(cid == 0)
    def _():
        pltpu.sync_copy(u_v, u_out.at[pl.ds(r0, RPT)])

    plsc.subcore_barrier()
    _edge_ring(utab, edges_hbm, sidx, didx, rows, gsem, ssem, acc, wid)
    plsc.subcore_barrier()
    pltpu.sync_copy(acc.at[pl.ds(sid * RPT, RPT)],
                    part_out.at[cid, pl.ds(sid * RPT, RPT)])


@functools.partial(
    pl.kernel,
    out_type=jax.ShapeDtypeStruct((NPAD,), jnp.float32),
    mesh=_MESH,
    scratch_types=[
        pltpu.VMEM((RPW, D), jnp.float32),
        pltpu.VMEM((RPW, D), jnp.float32),
        pltpu.VMEM((RPW, D), jnp.float32),
        pltpu.VMEM((RPW,), jnp.float32),
        pltpu.VMEM((D, D), jnp.float32),
        pltpu.VMEM((16,), jnp.float32),
        pltpu.VMEM((16,), jnp.float32),
        pltpu.VMEM((RPW,), jnp.float32),
    ],
    compiler_params=_SC_PARAMS,
)
def _comb_kernel(s2p_hbm, u_hbm, dinv_hbm, w2_hbm, b2_hbm, wl_hbm, out_hbm,
                 pa_v, pb_v, u_v, dinv_v, w2_v, b2_v, wl_v, out_v):
    cid = lax.axis_index("c")
    sid = lax.axis_index("s")
    wid = sid * NC + cid
    r0 = wid * RPW
    pltpu.sync_copy(s2p_hbm.at[0, pl.ds(r0, RPW)], pa_v)
    pltpu.sync_copy(s2p_hbm.at[1, pl.ds(r0, RPW)], pb_v)
    pltpu.sync_copy(u_hbm.at[pl.ds(r0, RPW)], u_v)
    pltpu.sync_copy(dinv_hbm.at[pl.ds(r0, RPW)], dinv_v)
    pltpu.sync_copy(w2_hbm, w2_v)
    pltpu.sync_copy(b2_hbm, b2_v)
    pltpu.sync_copy(wl_hbm, wl_v)

    lane = lax.iota(jnp.int32, 16)

    def grp(i, carry):
        dv = dinv_v[pl.ds(i * 16, 16)]
        z = jnp.zeros((16,), jnp.float32)
        for k in range(16):
            r = i * 16 + k
            d = lax.broadcast_in_dim(dv[k], (16,), ())
            v = d * (pa_v[r] + pb_v[r] + u_v[r])
            h = b2_v[...]
            for m in range(D):
                h = h + lax.broadcast_in_dim(v[m], (16,), ()) * w2_v[m]
            t = jnp.maximum(h, 0.0) * wl_v[...]
            s = lax.broadcast_in_dim(jnp.sum(t), (16,), ())
            z = jnp.where(lane == k, s, z)
        out_v[pl.ds(i * 16, 16)] = z
        return carry

    lax.fori_loop(0, RPW // 16, grp, 0)
    pltpu.sync_copy(out_v, out_hbm.at[pl.ds(r0, RPW)])


def _tc1_body(x_ref, w1_ref, deg_ref, g1_ref, dinv_ref):
    deg = jnp.sum(deg_ref[...], axis=0) + 1.0
    dinv = lax.rsqrt(deg)[:, None]
    h1 = jnp.dot(x_ref[...], w1_ref[...], preferred_element_type=jnp.float32)
    g1_ref[...] = h1 * dinv
    dinv_ref[...] = dinv[:, 0]


def kernel(x, edge_index, W1, b1, W2, b2, Wl, bl):
    edges = edge_index.astype(jnp.int32).reshape(2, NW, NJ, C)
    x_pad = jnp.pad(x, ((0, NPAD - N), (0, 0)))
    zrow = jnp.zeros((NPAD,), jnp.float32)
    zero16 = jnp.zeros((NPAD, D), jnp.float32)
    w2p = jnp.pad(W2, ((0, 0), (0, D - W2.shape[1])))
    b2p = jnp.pad(b2, (0, D - b2.shape[0]))
    wlp = jnp.pad(Wl[:, 0], (0, D - Wl.shape[0]))

    deg2 = _deg_kernel(edges, zrow)
    g1, dinv = pl.pallas_call(
        _tc1_body,
        out_shape=[jax.ShapeDtypeStruct((NPAD, D), jnp.float32),
                   jax.ShapeDtypeStruct((NPAD,), jnp.float32)],
    )(x_pad, W1, deg2)
    s1p = _agg1_kernel(g1, edges, zero16)
    s2p, u = _agg2_kernel(s1p, g1, dinv, b1, edges, zero16)
    out = _comb_kernel(s2p, u, dinv, w2p, b2p, wlp)
    return out[:N, None] + bl[None, :]
